# Initial kernel scaffold; baseline (speedup 1.0000x reference)
import jax
import jax.numpy as jnp
from jax.experimental import pallas as pl


def kernel(x, gate_w, w_gate, w_up, w_down):
    raise NotImplementedError("write your pallas kernel here")


# speedup vs baseline: 2.4165x; 2.4165x over previous
import functools

import jax
import jax.numpy as jnp
from jax.experimental import pallas as pl
from jax.experimental.pallas import tpu as pltpu

E = 8
K = 2
CAP = 512
NEG_INF = -1e30


def _router_body(x_ref, gw_ref, mslot_ref, p_ref, aux_ref, used_ref):
    x = x_ref[...]
    gw = gw_ref[...]
    T = x.shape[0]
    logits = jax.lax.dot_general(
        x, gw, (((1,), (1,)), ((), ())), preferred_element_type=jnp.float32)
    lane = jax.lax.broadcasted_iota(jnp.int32, (T, E), 1)
    m1 = jnp.max(logits, axis=1, keepdims=True)
    idx1 = jnp.min(jnp.where(logits == m1, lane, E), axis=1, keepdims=True)
    masked = jnp.where(lane == idx1, NEG_INF, logits)
    m2 = jnp.max(masked, axis=1, keepdims=True)
    idx2 = jnp.min(jnp.where(masked == m2, lane, E), axis=1, keepdims=True)
    t = jnp.exp(m2 - m1)
    w1 = 1.0 / (1.0 + t)
    w2 = t / (1.0 + t)
    mask = jnp.logical_or(lane == idx1, lane == idx2).astype(jnp.float32)
    inc = mask
    shift = 1
    while shift < T:
        shifted = jnp.concatenate(
            [jnp.zeros((shift, E), jnp.float32), inc[:T - shift]], axis=0)
        inc = inc + shifted
        shift *= 2
    slot = inc - mask
    kept = jnp.logical_and(mask > 0, slot < float(CAP))
    p = jnp.where(kept, jnp.where(lane == idx1, w1, w2), 0.0)
    mslot = jnp.where(kept, slot, -1.0)
    mslot_ref[...] = mslot
    p_ref[...] = p
    counts = jnp.sum(mask, axis=0, keepdims=True)
    frac = counts / float(T)
    mu = jnp.sum(frac, axis=1, keepdims=True) / float(E)
    var = jnp.sum((frac - mu) ** 2, axis=1, keepdims=True) / float(E - 1)
    aux = var * float(E)
    aux_ref[...] = jnp.broadcast_to(aux, (1, 128))
    used = jnp.sum((counts > 0).astype(jnp.int32), axis=1, keepdims=True)
    used_ref[...] = jnp.broadcast_to(used, (1, 128))


def _ffn_body(x_ref, mslot_ref, p_ref, wg_ref, wu_ref, wd_ref, out_ref,
              onehot_ref, xin_ref, acc_ref, *, n_itile):
    e = pl.program_id(0)
    it = pl.program_id(1)
    T = x_ref.shape[0]

    @pl.when(it == 0)
    def _dispatch():
        lane = jax.lax.broadcasted_iota(jnp.int32, (T, E), 1)
        sel = (lane == e).astype(jnp.float32)
        slot_col = jnp.sum(mslot_ref[...] * sel, axis=1, keepdims=True)
        cap_iota = jax.lax.broadcasted_iota(
            jnp.int32, (1, CAP), 1).astype(jnp.float32)
        onehot = (slot_col == cap_iota).astype(jnp.bfloat16)
        onehot_ref[...] = onehot
        xin_ref[...] = jax.lax.dot_general(
            onehot, x_ref[...].astype(jnp.bfloat16),
            (((0,), (0,)), ((), ())),
            preferred_element_type=jnp.float32,
        ).astype(jnp.bfloat16)

    xin = xin_ref[...]
    wg = wg_ref[0].astype(jnp.bfloat16)
    wu = wu_ref[0].astype(jnp.bfloat16)
    g = jax.lax.dot_general(xin, wg, (((1,), (1,)), ((), ())),
                            preferred_element_type=jnp.float32)
    u = jax.lax.dot_general(xin, wu, (((1,), (1,)), ((), ())),
                            preferred_element_type=jnp.float32)
    g = g / (1.0 + jnp.exp(-g))
    h = (g * u).astype(jnp.bfloat16)
    wd = wd_ref[0].astype(jnp.bfloat16)
    contrib = jax.lax.dot_general(h, wd, (((1,), (1,)), ((), ())),
                                  preferred_element_type=jnp.float32)

    @pl.when(it == 0)
    def _init_acc():
        acc_ref[...] = contrib

    @pl.when(it > 0)
    def _add_acc():
        acc_ref[...] = acc_ref[...] + contrib

    @pl.when(it == n_itile - 1)
    def _combine():
        lane = jax.lax.broadcasted_iota(jnp.int32, (T, E), 1)
        sel = (lane == e).astype(jnp.float32)
        p_col = jnp.sum(p_ref[...] * sel, axis=1, keepdims=True)
        oh_w = onehot_ref[...].astype(jnp.float32) * p_col
        out_contrib = jnp.dot(oh_w, acc_ref[...],
                              preferred_element_type=jnp.float32)

        @pl.when(e == 0)
        def _():
            out_ref[...] = out_contrib

        @pl.when(e > 0)
        def _():
            out_ref[...] = out_ref[...] + out_contrib


def kernel(x, gate_w, w_gate, w_up, w_down):
    B, S, H = x.shape
    T = B * S
    I = w_gate.shape[1]
    x_flat = x.reshape(T, H)

    mslot, p, aux, used = pl.pallas_call(
        _router_body,
        out_shape=(
            jax.ShapeDtypeStruct((T, E), jnp.float32),
            jax.ShapeDtypeStruct((T, E), jnp.float32),
            jax.ShapeDtypeStruct((1, 128), jnp.float32),
            jax.ShapeDtypeStruct((1, 128), jnp.int32),
        ),
    )(x_flat, gate_w)

    IT = 1024
    n_itile = I // IT
    out = pl.pallas_call(
        functools.partial(_ffn_body, n_itile=n_itile),
        grid=(E, n_itile),
        in_specs=[
            pl.BlockSpec((T, H), lambda e, it: (0, 0)),
            pl.BlockSpec((T, E), lambda e, it: (0, 0)),
            pl.BlockSpec((T, E), lambda e, it: (0, 0)),
            pl.BlockSpec((1, IT, H), lambda e, it: (e, it, 0)),
            pl.BlockSpec((1, IT, H), lambda e, it: (e, it, 0)),
            pl.BlockSpec((1, H, IT), lambda e, it: (e, 0, it)),
        ],
        out_specs=pl.BlockSpec((T, H), lambda e, it: (0, 0)),
        out_shape=jax.ShapeDtypeStruct((T, H), jnp.float32),
        scratch_shapes=[
            pltpu.VMEM((T, CAP), jnp.bfloat16),
            pltpu.VMEM((CAP, H), jnp.bfloat16),
            pltpu.VMEM((CAP, H), jnp.float32),
        ],
        compiler_params=pltpu.CompilerParams(
            dimension_semantics=("arbitrary", "arbitrary"),
            vmem_limit_bytes=62 * 1024 * 1024,
        ),
    )(x_flat, mslot, p, w_gate, w_up, w_down)

    return (out.reshape(B, S, H), aux[0, 0], used[0, 0])
